# baseline (device time: 8940 ns/iter reference)
import jax
import jax.numpy as jnp
from jax import lax
from jax.experimental import pallas as pl
from jax.experimental.pallas import tpu as pltpu

N_GLOBAL_FEATURES = 1024
EPS = 1e-5


def kernel(x, gamma, beta):
    m, n = x.shape

    def body(x_ref, g_ref, b_ref, out_ref, stats_send, stats_recv,
             send_sem, recv_sem):
        my_x = lax.axis_index("x")
        my_y = lax.axis_index("y")
        peer = (my_x, 1 - my_y)

        barrier_sem = pltpu.get_barrier_semaphore()
        pl.semaphore_signal(barrier_sem, inc=1, device_id=peer,
                            device_id_type=pl.DeviceIdType.MESH)

        xb = x_ref[:, :].astype(jnp.bfloat16)
        ones_row = jnp.ones((1, n), jnp.bfloat16)
        dnums = (((1,), (1,)), ((), ()))
        s_lane = lax.dot_general(ones_row, xb, dnums,
                                 preferred_element_type=jnp.float32)
        sq_lane = lax.dot_general(ones_row, xb * xb, dnums,
                                  preferred_element_type=jnp.float32)
        stats_send[0:1, :] = s_lane
        stats_send[1:2, :] = sq_lane

        pl.semaphore_wait(barrier_sem, 1)

        rdma = pltpu.make_async_remote_copy(
            src_ref=stats_send,
            dst_ref=stats_recv,
            send_sem=send_sem,
            recv_sem=recv_sem,
            device_id=peer,
            device_id_type=pl.DeviceIdType.MESH,
        )
        rdma.start()

        g = g_ref[:, :].astype(jnp.bfloat16)
        b = b_ref[:, :].astype(jnp.bfloat16)
        xg = xb * g

        rdma.wait_recv()

        total = stats_send[0:1, :] + stats_recv[0:1, :]
        total_sq = stats_send[1:2, :] + stats_recv[1:2, :]
        mean_l = total / N_GLOBAL_FEATURES
        var_l = total_sq / N_GLOBAL_FEATURES - mean_l * mean_l
        inv_l = lax.rsqrt(var_l + EPS)
        mi_l = mean_l * inv_l
        inv = jnp.concatenate(
            [inv_l[:, i * 128:(i + 1) * 128].reshape(128, 1)
             for i in range(8)], axis=0).astype(jnp.bfloat16)
        mi = jnp.concatenate(
            [mi_l[:, i * 128:(i + 1) * 128].reshape(128, 1)
             for i in range(8)], axis=0).astype(jnp.bfloat16)
        out_ref[:, :] = xg * inv - mi * g + b

        rdma.wait_send()

    return pl.pallas_call(
        body,
        out_shape=jax.ShapeDtypeStruct((m, n), jnp.bfloat16),
        in_specs=[
            pl.BlockSpec(memory_space=pltpu.VMEM),
            pl.BlockSpec(memory_space=pltpu.VMEM),
            pl.BlockSpec(memory_space=pltpu.VMEM),
        ],
        out_specs=pl.BlockSpec(memory_space=pltpu.VMEM),
        scratch_shapes=[
            pltpu.VMEM((2, 1024), jnp.float32),
            pltpu.VMEM((2, 1024), jnp.float32),
            pltpu.SemaphoreType.DMA,
            pltpu.SemaphoreType.DMA,
        ],
        compiler_params=pltpu.CompilerParams(collective_id=0),
    )(x, gamma.reshape(1, n), beta.reshape(1, n))


# device time: 8622 ns/iter; 1.0369x vs baseline; 1.0369x over previous
import jax
import jax.numpy as jnp
from jax import lax
from jax.experimental import pallas as pl
from jax.experimental.pallas import tpu as pltpu

N_GLOBAL_FEATURES = 1024
EPS = 1e-5


def kernel(x, gamma, beta):
    m, n = x.shape
    half = m // 2

    def body(x_ref, g_ref, b_ref, out_ref, stats_send, stats_recv,
             send_sems, recv_sems):
        my_x = lax.axis_index("x")
        my_y = lax.axis_index("y")
        peer = (my_x, 1 - my_y)

        barrier_sem = pltpu.get_barrier_semaphore()
        pl.semaphore_signal(barrier_sem, inc=1, device_id=peer,
                            device_id_type=pl.DeviceIdType.MESH)

        xv = x_ref[:, :].astype(jnp.float32)

        x0 = xv[0:half, :]
        stats_send[0:4, :] = jnp.sum(x0, axis=1, keepdims=True).reshape(4, 128)
        stats_send[4:8, :] = jnp.sum(x0 * x0, axis=1,
                                     keepdims=True).reshape(4, 128)

        pl.semaphore_wait(barrier_sem, 1)

        rdma0 = pltpu.make_async_remote_copy(
            src_ref=stats_send.at[0:8],
            dst_ref=stats_recv.at[0:8],
            send_sem=send_sems.at[0],
            recv_sem=recv_sems.at[0],
            device_id=peer,
            device_id_type=pl.DeviceIdType.MESH,
        )
        rdma0.start()

        x1 = xv[half:m, :]
        stats_send[8:12, :] = jnp.sum(x1, axis=1, keepdims=True).reshape(4, 128)
        stats_send[12:16, :] = jnp.sum(x1 * x1, axis=1,
                                       keepdims=True).reshape(4, 128)

        rdma1 = pltpu.make_async_remote_copy(
            src_ref=stats_send.at[8:16],
            dst_ref=stats_recv.at[8:16],
            send_sem=send_sems.at[1],
            recv_sem=recv_sems.at[1],
            device_id=peer,
            device_id_type=pl.DeviceIdType.MESH,
        )
        rdma1.start()

        g = g_ref[:, :].astype(jnp.bfloat16)
        b = b_ref[:, :].astype(jnp.bfloat16)
        xg = xv.astype(jnp.bfloat16) * g

        def normalize(lo, srow):
            total = stats_send[srow:srow + 4, :] + stats_recv[srow:srow + 4, :]
            total_sq = (stats_send[srow + 4:srow + 8, :]
                        + stats_recv[srow + 4:srow + 8, :])
            mean4 = total / N_GLOBAL_FEATURES
            var4 = total_sq / N_GLOBAL_FEATURES - mean4 * mean4
            inv4 = lax.rsqrt(var4 + EPS)
            mi4 = mean4 * inv4
            inv = jnp.concatenate(
                [inv4[i:i + 1, :].reshape(128, 1) for i in range(4)],
                axis=0).astype(jnp.bfloat16)
            mi = jnp.concatenate(
                [mi4[i:i + 1, :].reshape(128, 1) for i in range(4)],
                axis=0).astype(jnp.bfloat16)
            out_ref[lo:lo + half, :] = xg[lo:lo + half, :] * inv - mi * g + b

        rdma0.wait_recv()
        normalize(0, 0)
        rdma1.wait_recv()
        normalize(half, 8)

        rdma0.wait_send()
        rdma1.wait_send()

    return pl.pallas_call(
        body,
        out_shape=jax.ShapeDtypeStruct((m, n), jnp.bfloat16),
        in_specs=[
            pl.BlockSpec(memory_space=pltpu.VMEM),
            pl.BlockSpec(memory_space=pltpu.VMEM),
            pl.BlockSpec(memory_space=pltpu.VMEM),
        ],
        out_specs=pl.BlockSpec(memory_space=pltpu.VMEM),
        scratch_shapes=[
            pltpu.VMEM((16, 128), jnp.float32),
            pltpu.VMEM((16, 128), jnp.float32),
            pltpu.SemaphoreType.DMA((2,)),
            pltpu.SemaphoreType.DMA((2,)),
        ],
        compiler_params=pltpu.CompilerParams(collective_id=0),
    )(x, gamma.reshape(1, n), beta.reshape(1, n))
